# baseline (speedup 1.0000x reference)
import jax
import jax.numpy as jnp
from jax.experimental import pallas as pl

N = 10000
ROWS = 200


def _hop_kernel(adj_ref, x_ref, o_ref):
    o_ref[...] = jnp.dot(adj_ref[...], x_ref[...],
                         preferred_element_type=jnp.float32)


def _hop_mlp_kernel(adj_ref, h_ref, W1_ref, b1_ref, W2_ref, b2_ref, o_ref):
    h2 = jnp.dot(adj_ref[...], h_ref[...], preferred_element_type=jnp.float32)
    h = jnp.dot(h2, W1_ref[...], preferred_element_type=jnp.float32) + b1_ref[...]
    h = jnp.maximum(h, 0.0)
    z = jnp.dot(h, W2_ref[...], preferred_element_type=jnp.float32) + b2_ref[...]
    m = jnp.max(z, axis=1, keepdims=True)
    zs = z - m
    lse = jnp.log(jnp.sum(jnp.exp(zs), axis=1, keepdims=True))
    o_ref[...] = zs - lse


def kernel(x, adj, W1, b1, W2, b2):
    nfeat = x.shape[1]
    nclass = W2.shape[1]
    grid = (N // ROWS,)

    adj_spec = pl.BlockSpec((ROWS, N), lambda i: (i, 0))
    full = lambda shape: pl.BlockSpec(shape, lambda i: (0, 0))

    h1 = pl.pallas_call(
        _hop_kernel,
        grid=grid,
        in_specs=[adj_spec, full((N, nfeat))],
        out_specs=pl.BlockSpec((ROWS, nfeat), lambda i: (i, 0)),
        out_shape=jax.ShapeDtypeStruct((N, nfeat), jnp.float32),
    )(adj, x)

    b1r = b1.reshape(1, -1)
    b2r = b2.reshape(1, -1)
    out = pl.pallas_call(
        _hop_mlp_kernel,
        grid=grid,
        in_specs=[
            adj_spec,
            full((N, nfeat)),
            full(W1.shape),
            full(b1r.shape),
            full(W2.shape),
            full(b2r.shape),
        ],
        out_specs=pl.BlockSpec((ROWS, nclass), lambda i: (i, 0)),
        out_shape=jax.ShapeDtypeStruct((N, nclass), jnp.float32),
    )(adj, h1, W1, b1r, W2, b2r)
    return out
